# Initial kernel scaffold; baseline (speedup 1.0000x reference)
import jax
import jax.numpy as jnp
from jax.experimental import pallas as pl


def kernel(ufeats, ifeats, P_w, P_b, Q_w, Q_b, edge_index):
    raise NotImplementedError("write your pallas kernel here")


# speedup vs baseline: 4.2691x; 4.2691x over previous
import functools

import jax
import jax.numpy as jnp
from jax import lax
from jax.experimental import pallas as pl
from jax.experimental.pallas import tpu as pltpu
from jax.experimental.pallas import tpu_sc as plsc

_D = 128
_NB = 2
_DW = _D * _NB
_NCLS = 5
_NCORES = 2
_NSUB = 16
_NW = _NCORES * _NSUB
_CHUNK = 80


def _uh_matmul(ufeats, w, b):
    n_users = ufeats.shape[0]
    blk = 1000
    grid = n_users // blk

    def body(u_ref, w_ref, b_ref, o_ref):
        o_ref[...] = (
            jnp.dot(u_ref[...], w_ref[...], preferred_element_type=jnp.float32)
            + b_ref[...]
        )

    return pl.pallas_call(
        body,
        grid=(grid,),
        in_specs=[
            pl.BlockSpec((blk, _D), lambda i: (i, 0)),
            pl.BlockSpec((_D, _DW), lambda i: (0, 0)),
            pl.BlockSpec((1, _DW), lambda i: (0, 0)),
        ],
        out_specs=pl.BlockSpec((blk, _DW), lambda i: (i, 0)),
        out_shape=jax.ShapeDtypeStruct((n_users, _DW), jnp.float32),
    )(ufeats, w, b)


def _edge_logits_sc(uh, ifeats, src, dst, qcoef):
    n_edges = src.shape[0]
    epw = n_edges // _NW
    nchunk = epw // _CHUNK
    mesh = plsc.VectorSubcoreMesh(core_axis_name="c", subcore_axis_name="s")

    @functools.partial(
        pl.kernel,
        out_type=jax.ShapeDtypeStruct((n_edges * _NCLS,), jnp.float32),
        mesh=mesh,
        scratch_types=[
            pltpu.VMEM((16,), jnp.float32),
            pltpu.VMEM((epw,), jnp.int32),
            pltpu.VMEM((epw,), jnp.int32),
            pltpu.VMEM((_CHUNK, _DW), jnp.float32),
            pltpu.VMEM((_CHUNK, _D), jnp.float32),
            pltpu.VMEM((_CHUNK * _NCLS,), jnp.float32),
            pltpu.VMEM((_CHUNK, _DW), jnp.float32),
            pltpu.VMEM((_CHUNK, _D), jnp.float32),
            pltpu.VMEM((_CHUNK * _NCLS,), jnp.float32),
            pltpu.SemaphoreType.DMA,
            pltpu.SemaphoreType.DMA,
            pltpu.SemaphoreType.DMA,
            pltpu.SemaphoreType.DMA,
        ],
        compiler_params=pltpu.CompilerParams(needs_layout_passes=False),
    )
    def k(
        uh_hbm,
        if_hbm,
        src_hbm,
        dst_hbm,
        q_hbm,
        out_hbm,
        q_v,
        src_v,
        dst_v,
        u_a,
        i_a,
        o_a,
        u_b,
        i_b,
        o_b,
        sem_ua,
        sem_ia,
        sem_ub,
        sem_ib,
    ):
        wid = lax.axis_index("s") * _NCORES + lax.axis_index("c")
        base = wid * epw
        pltpu.sync_copy(q_hbm, q_v)
        pltpu.sync_copy(src_hbm.at[pl.ds(base, epw)], src_v)
        pltpu.sync_copy(dst_hbm.at[pl.ds(base, epw)], dst_v)
        qvec = q_v[pl.ds(0, 16)]
        q = [qvec[i] for i in range(3 * _NCLS)]
        lane = lax.iota(jnp.int32, 16)
        lane5 = lane * _NCLS

        def issue(c, u_v, i_v, sem_u, sem_i):
            off = c * _CHUNK
            pltpu.async_copy(uh_hbm.at[src_v.at[pl.ds(off, _CHUNK)]], u_v, sem_u)
            pltpu.async_copy(if_hbm.at[dst_v.at[pl.ds(off, _CHUNK)]], i_v, sem_i)

        def finish(c, u_v, i_v, o_v, sem_u, sem_i):
            off = c * _CHUNK
            pltpu.make_async_copy(
                uh_hbm.at[src_v.at[pl.ds(off, _CHUNK)]], u_v, sem_u
            ).wait()
            pltpu.make_async_copy(
                if_hbm.at[dst_v.at[pl.ds(off, _CHUNK)]], i_v, sem_i
            ).wait()

            @plsc.parallel_loop(0, _CHUNK // 16)
            def _(g):
                r0 = jnp.zeros((16,), jnp.float32)
                r1 = jnp.zeros((16,), jnp.float32)
                for l in range(16):
                    e = g * 16 + l
                    iv = i_v[e, pl.ds(0, 16)]
                    acc0 = u_v[e, pl.ds(0, 16)] * iv
                    acc1 = u_v[e, pl.ds(_D, 16)] * iv
                    for t in range(1, _D // 16):
                        s0 = 16 * t
                        iv = i_v[e, pl.ds(s0, 16)]
                        acc0 = acc0 + u_v[e, pl.ds(s0, 16)] * iv
                        acc1 = acc1 + u_v[e, pl.ds(_D + s0, 16)] * iv
                    r0 = jnp.where(lane == l, jnp.sum(acc0), r0)
                    r1 = jnp.where(lane == l, jnp.sum(acc1), r1)
                bi = lane5 + g * (16 * _NCLS)
                for c in range(_NCLS):
                    lv = r0 * q[3 * c] + r1 * q[3 * c + 1] + q[3 * c + 2]
                    plsc.store_scatter(o_v, [bi + c], lv)

            pltpu.sync_copy(
                o_v, out_hbm.at[pl.ds((base + off) * _NCLS, _CHUNK * _NCLS)]
            )

        issue(0, u_a, i_a, sem_ua, sem_ia)

        def loop_body(j, carry):
            c = 2 * j
            issue(c + 1, u_b, i_b, sem_ub, sem_ib)
            finish(c, u_a, i_a, o_a, sem_ua, sem_ia)
            issue(c + 2, u_a, i_a, sem_ua, sem_ia)
            finish(c + 1, u_b, i_b, o_b, sem_ub, sem_ib)
            return carry

        lax.fori_loop(0, (nchunk - 1) // 2, loop_body, 0)
        finish(nchunk - 1, u_a, i_a, o_a, sem_ua, sem_ia)

    return k(uh, ifeats, src, dst, qcoef)


def kernel(ufeats, ifeats, P_w, P_b, Q_w, Q_b, edge_index):
    w = jnp.transpose(P_w, (2, 0, 1)).reshape(_D, _DW)
    b = P_b.reshape(1, _DW)
    uh = _uh_matmul(ufeats, w, b)
    src = edge_index[0]
    dst = edge_index[1]
    qcoef = jnp.concatenate(
        [
            jnp.stack([Q_w[:, 0], Q_w[:, 1], Q_b], axis=1).reshape(-1),
            jnp.zeros((1,), jnp.float32),
        ]
    )
    n_edges = src.shape[0]
    flat = _edge_logits_sc(uh, ifeats, src, dst, qcoef)
    return flat.reshape(n_edges, _NCLS)


# speedup vs baseline: 4.4694x; 1.0469x over previous
import functools

import jax
import jax.numpy as jnp
from jax import lax
from jax.experimental import pallas as pl
from jax.experimental.pallas import tpu as pltpu
from jax.experimental.pallas import tpu_sc as plsc

_D = 128
_NB = 2
_DW = _D * _NB
_NCLS = 5
_NCORES = 2
_NSUB = 16
_NW = _NCORES * _NSUB
_CHUNK = 128


def _uh_matmul(ufeats, p_w, p_b):
    n_users = ufeats.shape[0]
    blk = 1000
    grid = n_users // blk

    def body(u_ref, w_ref, b_ref, o_ref):
        u = u_ref[...]
        w = w_ref[...]
        o_ref[:, 0:_D] = (
            lax.dot_general(
                u, w[0], (((1,), (1,)), ((), ())), preferred_element_type=jnp.float32
            )
            + b_ref[0:1, :]
        )
        o_ref[:, _D:_DW] = (
            lax.dot_general(
                u, w[1], (((1,), (1,)), ((), ())), preferred_element_type=jnp.float32
            )
            + b_ref[1:2, :]
        )

    return pl.pallas_call(
        body,
        grid=(grid,),
        in_specs=[
            pl.BlockSpec((blk, _D), lambda i: (i, 0)),
            pl.BlockSpec((_NB, _D, _D), lambda i: (0, 0, 0)),
            pl.BlockSpec((_NB, _D), lambda i: (0, 0)),
        ],
        out_specs=pl.BlockSpec((blk, _DW), lambda i: (i, 0)),
        out_shape=jax.ShapeDtypeStruct((n_users, _DW), jnp.float32),
    )(ufeats, p_w, p_b)


def _edge_scores_sc(uh, ifeats, edge_index):
    n_edges = edge_index.shape[1]
    nchunk = n_edges // _CHUNK
    npair = (nchunk // _NW) // 2
    nrem = nchunk - _NW * 2 * npair
    mesh = plsc.VectorSubcoreMesh(core_axis_name="c", subcore_axis_name="s")

    @functools.partial(
        pl.kernel,
        out_type=jax.ShapeDtypeStruct((_NB, n_edges), jnp.float32),
        mesh=mesh,
        scratch_types=[
            pltpu.VMEM((2, _CHUNK), jnp.int32),
            pltpu.VMEM((_CHUNK, _DW), jnp.float32),
            pltpu.VMEM((_CHUNK, _D), jnp.float32),
            pltpu.VMEM((_NB, _CHUNK), jnp.float32),
            pltpu.VMEM((2, _CHUNK), jnp.int32),
            pltpu.VMEM((_CHUNK, _DW), jnp.float32),
            pltpu.VMEM((_CHUNK, _D), jnp.float32),
            pltpu.VMEM((_NB, _CHUNK), jnp.float32),
            pltpu.SemaphoreType.DMA,
            pltpu.SemaphoreType.DMA,
            pltpu.SemaphoreType.DMA,
            pltpu.SemaphoreType.DMA,
        ],
        compiler_params=pltpu.CompilerParams(needs_layout_passes=False),
    )
    def k(
        uh_hbm,
        if_hbm,
        ei_hbm,
        out_hbm,
        x_a,
        u_a,
        i_a,
        o_a,
        x_b,
        u_b,
        i_b,
        o_b,
        sem_ua,
        sem_ia,
        sem_ub,
        sem_ib,
    ):
        wid = lax.axis_index("s") * _NCORES + lax.axis_index("c")
        lane = lax.iota(jnp.int32, 16)

        def issue(c, x_v, u_v, i_v, sem_u, sem_i):
            off = c * _CHUNK
            pltpu.sync_copy(ei_hbm.at[:, pl.ds(off, _CHUNK)], x_v)
            pltpu.async_copy(uh_hbm.at[x_v.at[0]], u_v, sem_u)
            pltpu.async_copy(if_hbm.at[x_v.at[1]], i_v, sem_i)

        def finish(c, x_v, u_v, i_v, o_v, sem_u, sem_i):
            off = c * _CHUNK
            pltpu.make_async_copy(uh_hbm.at[x_v.at[0]], u_v, sem_u).wait()
            pltpu.make_async_copy(if_hbm.at[x_v.at[1]], i_v, sem_i).wait()

            @plsc.parallel_loop(0, _CHUNK // 16)
            def _(g):
                r0 = jnp.zeros((16,), jnp.float32)
                r1 = jnp.zeros((16,), jnp.float32)
                for l in range(16):
                    e = g * 16 + l
                    iv = i_v[e, pl.ds(0, 16)]
                    acc0 = u_v[e, pl.ds(0, 16)] * iv
                    acc1 = u_v[e, pl.ds(_D, 16)] * iv
                    for t in range(1, _D // 16):
                        s0 = 16 * t
                        iv = i_v[e, pl.ds(s0, 16)]
                        acc0 = acc0 + u_v[e, pl.ds(s0, 16)] * iv
                        acc1 = acc1 + u_v[e, pl.ds(_D + s0, 16)] * iv
                    r0 = jnp.where(lane == l, jnp.sum(acc0), r0)
                    r1 = jnp.where(lane == l, jnp.sum(acc1), r1)
                o_v[0, pl.ds(g * 16, 16)] = r0
                o_v[1, pl.ds(g * 16, 16)] = r1

            pltpu.sync_copy(o_v, out_hbm.at[:, pl.ds(off, _CHUNK)])

        issue(wid, x_a, u_a, i_a, sem_ua, sem_ia)

        def loop_body(j, carry):
            c = (2 * j) * _NW + wid
            issue(c + _NW, x_b, u_b, i_b, sem_ub, sem_ib)
            finish(c, x_a, u_a, i_a, o_a, sem_ua, sem_ia)

            @pl.when(j + 1 < npair)
            def _():
                issue(c + 2 * _NW, x_a, u_a, i_a, sem_ua, sem_ia)

            @pl.when(jnp.logical_and(j + 1 == npair, wid < nrem))
            def _():
                issue(2 * npair * _NW + wid, x_a, u_a, i_a, sem_ua, sem_ia)

            finish(c + _NW, x_b, u_b, i_b, o_b, sem_ub, sem_ib)
            return carry

        lax.fori_loop(0, npair, loop_body, 0)

        @pl.when(wid < nrem)
        def _():
            finish(2 * npair * _NW + wid, x_a, u_a, i_a, o_a, sem_ua, sem_ia)

    return k(uh, ifeats, edge_index)


def _project(r, q_w, qb):
    nb, n_edges = r.shape
    n_cls = q_w.shape[0]
    blk = 12800
    grid = n_edges // blk

    def body(r_ref, qw_ref, qb_ref, o_ref):
        o_ref[...] = (
            lax.dot_general(
                r_ref[...],
                qw_ref[...],
                (((0,), (1,)), ((), ())),
                preferred_element_type=jnp.float32,
            )
            + qb_ref[...]
        )

    return pl.pallas_call(
        body,
        grid=(grid,),
        in_specs=[
            pl.BlockSpec((nb, blk), lambda i: (0, i)),
            pl.BlockSpec((n_cls, nb), lambda i: (0, 0)),
            pl.BlockSpec((1, n_cls), lambda i: (0, 0)),
        ],
        out_specs=pl.BlockSpec((blk, n_cls), lambda i: (i, 0)),
        out_shape=jax.ShapeDtypeStruct((n_edges, n_cls), jnp.float32),
    )(r, q_w, qb)


def kernel(ufeats, ifeats, P_w, P_b, Q_w, Q_b, edge_index):
    uh = _uh_matmul(ufeats, P_w, P_b)
    r = _edge_scores_sc(uh, ifeats, edge_index)
    return _project(r, Q_w, Q_b.reshape(1, _NCLS))


# speedup vs baseline: 10.9653x; 2.4534x over previous
import functools

import jax
import jax.numpy as jnp
from jax import lax
from jax.experimental import pallas as pl
from jax.experimental.pallas import tpu as pltpu
from jax.experimental.pallas import tpu_sc as plsc

_D = 128
_NB = 2
_DW = _D * _NB
_NCLS = 5
_NCORES = 2
_NSUB = 16
_NW = _NCORES * _NSUB
_CHUNK = 128


def _uh_matmul(ufeats, w_lo, w_hi, b_lo, b_hi):
    n_users = ufeats.shape[0]
    blk = 1000
    grid = n_users // blk

    def body(u_ref, wl_ref, wh_ref, bl_ref, bh_ref, o_ref):
        u = u_ref[...]
        lo = (
            jnp.dot(u, wl_ref[...], preferred_element_type=jnp.float32) + bl_ref[...]
        ).astype(jnp.bfloat16)
        hi = (
            jnp.dot(u, wh_ref[...], preferred_element_type=jnp.float32) + bh_ref[...]
        ).astype(jnp.bfloat16)
        lo32 = lax.bitcast_convert_type(lo, jnp.uint16).astype(jnp.uint32)
        hi32 = lax.bitcast_convert_type(hi, jnp.uint16).astype(jnp.uint32)
        o_ref[...] = lax.bitcast_convert_type(lo32 | (hi32 << 16), jnp.int32)

    return pl.pallas_call(
        body,
        grid=(grid,),
        in_specs=[
            pl.BlockSpec((blk, _D), lambda i: (i, 0)),
            pl.BlockSpec((_D, _D), lambda i: (0, 0)),
            pl.BlockSpec((_D, _D), lambda i: (0, 0)),
            pl.BlockSpec((1, _D), lambda i: (0, 0)),
            pl.BlockSpec((1, _D), lambda i: (0, 0)),
        ],
        out_specs=pl.BlockSpec((blk, _D), lambda i: (i, 0)),
        out_shape=jax.ShapeDtypeStruct((n_users, _D), jnp.int32),
    )(ufeats, w_lo, w_hi, b_lo, b_hi)


def _edge_logits_sc(uh, ifeats, edge_index, qcoef):
    n_edges = edge_index.shape[1]
    nchunk = n_edges // _CHUNK
    npair = (nchunk // _NW) // 2
    nrem = nchunk - _NW * 2 * npair
    mesh = plsc.VectorSubcoreMesh(core_axis_name="c", subcore_axis_name="s")

    @functools.partial(
        pl.kernel,
        out_type=jax.ShapeDtypeStruct((_NCLS, n_edges), jnp.float32),
        mesh=mesh,
        scratch_types=[
            pltpu.VMEM((48,), jnp.float32),
            pltpu.VMEM((2, _CHUNK), jnp.int32),
            pltpu.VMEM((_CHUNK, _D), jnp.int32),
            pltpu.VMEM((_CHUNK, _D), jnp.float32),
            pltpu.VMEM((16, _CHUNK), jnp.float32),
            pltpu.VMEM((2, _CHUNK), jnp.int32),
            pltpu.VMEM((_CHUNK, _D), jnp.int32),
            pltpu.VMEM((_CHUNK, _D), jnp.float32),
            pltpu.VMEM((16, _CHUNK), jnp.float32),
            pltpu.SemaphoreType.DMA,
            pltpu.SemaphoreType.DMA,
            pltpu.SemaphoreType.DMA,
            pltpu.SemaphoreType.DMA,
            pltpu.SemaphoreType.DMA,
            pltpu.SemaphoreType.DMA,
        ],
        compiler_params=pltpu.CompilerParams(needs_layout_passes=False),
    )
    def k(
        uh_hbm,
        if_hbm,
        ei_hbm,
        q_hbm,
        out_hbm,
        q_v,
        x_a,
        u_a,
        i_a,
        o_a,
        x_b,
        u_b,
        i_b,
        o_b,
        sem_xa,
        sem_ua,
        sem_ia,
        sem_xb,
        sem_ub,
        sem_ib,
    ):
        wid = lax.axis_index("s") * _NCORES + lax.axis_index("c")
        pltpu.sync_copy(q_hbm, q_v)
        qv0 = q_v[pl.ds(0, 16)]
        qv1 = q_v[pl.ds(16, 16)]
        qvb = q_v[pl.ds(32, 16)]
        lane = lax.iota(jnp.int32, 16)

        def fetch_idx(c, x_v, sem_x):
            pltpu.async_copy(ei_hbm.at[:, pl.ds(c * _CHUNK, _CHUNK)], x_v, sem_x)

        def gather(c, x_v, u_v, i_v, sem_x, sem_u, sem_i):
            pltpu.make_async_copy(
                ei_hbm.at[:, pl.ds(c * _CHUNK, _CHUNK)], x_v, sem_x
            ).wait()
            pltpu.async_copy(uh_hbm.at[x_v.at[0]], u_v, sem_u)
            pltpu.async_copy(if_hbm.at[x_v.at[1]], i_v, sem_i)

        def waitg(x_v, u_v, i_v, sem_u, sem_i):
            pltpu.make_async_copy(uh_hbm.at[x_v.at[0]], u_v, sem_u).wait()
            pltpu.make_async_copy(if_hbm.at[x_v.at[1]], i_v, sem_i).wait()

        def compute_store(c, u_v, i_v, o_v):
            off = c * _CHUNK

            @plsc.parallel_loop(0, _CHUNK, unroll=4)
            def _(e):
                acc0 = None
                acc1 = None
                for t in range(4):
                    iva = i_v[e, pl.ds(32 * t, 16)]
                    ivb = i_v[e, pl.ds(32 * t + 16, 16)]
                    ua0, ub0 = plsc.unpack(
                        plsc.bitcast(u_v[e, pl.ds(16 * t, 16)], jnp.bfloat16),
                        format=plsc.PackFormat.INTERLEAVED,
                    )
                    ua1, ub1 = plsc.unpack(
                        plsc.bitcast(u_v[e, pl.ds(64 + 16 * t, 16)], jnp.bfloat16),
                        format=plsc.PackFormat.INTERLEAVED,
                    )
                    c0 = ua0 * iva + ub0 * ivb
                    c1 = ua1 * iva + ub1 * ivb
                    acc0 = c0 if acc0 is None else acc0 + c0
                    acc1 = c1 if acc1 is None else acc1 + c1
                lv = qv0 * jnp.sum(acc0) + qv1 * jnp.sum(acc1) + qvb
                plsc.store_scatter(o_v, [lane, jnp.broadcast_to(e, (16,))], lv)

            pltpu.sync_copy(
                o_v.at[pl.ds(0, _NCLS)], out_hbm.at[:, pl.ds(off, _CHUNK)]
            )

        crem = 2 * npair * _NW + wid
        fetch_idx(wid, x_a, sem_xa)
        fetch_idx(_NW + wid, x_b, sem_xb)
        gather(wid, x_a, u_a, i_a, sem_xa, sem_ua, sem_ia)

        def loop_body(j, carry):
            c = (2 * j) * _NW + wid
            last = j + 1 == npair
            gather(c + _NW, x_b, u_b, i_b, sem_xb, sem_ub, sem_ib)
            waitg(x_a, u_a, i_a, sem_ua, sem_ia)

            @pl.when(jnp.logical_not(last))
            def _():
                fetch_idx(c + 2 * _NW, x_a, sem_xa)

            @pl.when(jnp.logical_and(last, wid < nrem))
            def _():
                fetch_idx(crem, x_a, sem_xa)

            compute_store(c, u_a, i_a, o_a)

            @pl.when(jnp.logical_not(last))
            def _():
                gather(c + 2 * _NW, x_a, u_a, i_a, sem_xa, sem_ua, sem_ia)

            @pl.when(jnp.logical_and(last, wid < nrem))
            def _():
                gather(crem, x_a, u_a, i_a, sem_xa, sem_ua, sem_ia)

            waitg(x_b, u_b, i_b, sem_ub, sem_ib)

            @pl.when(jnp.logical_not(last))
            def _():
                fetch_idx(c + 3 * _NW, x_b, sem_xb)

            compute_store(c + _NW, u_b, i_b, o_b)
            return carry

        lax.fori_loop(0, npair, loop_body, 0)

        @pl.when(wid < nrem)
        def _():
            waitg(x_a, u_a, i_a, sem_ua, sem_ia)
            compute_store(crem, u_a, i_a, o_a)

    return k(uh, ifeats, edge_index, qcoef)


def kernel(ufeats, ifeats, P_w, P_b, Q_w, Q_b, edge_index):
    w = jnp.transpose(P_w, (2, 0, 1)).reshape(_D, _DW)
    b = P_b.reshape(_DW)
    c = jnp.arange(_D)
    ll = c // 16
    base = jnp.where(ll < 4, 32 * ll, _D + 32 * (ll - 4)) + (c % 16)
    uh = _uh_matmul(
        ufeats,
        w[:, base],
        w[:, base + 16],
        b[base].reshape(1, _D),
        b[base + 16].reshape(1, _D),
    )
    pad = jnp.zeros((11,), jnp.float32)
    qcoef = jnp.concatenate([Q_w[:, 0], pad, Q_w[:, 1], pad, Q_b, pad])
    out5 = _edge_logits_sc(uh, ifeats, edge_index, qcoef)
    return out5.T
